# Initial kernel scaffold; baseline (speedup 1.0000x reference)
import jax
import jax.numpy as jnp
from jax.experimental import pallas as pl


def kernel(x, edge_index, batch, W1, b1, W2, b2):
    raise NotImplementedError("write your pallas kernel here")


# speedup vs baseline: 19.1981x; 19.1981x over previous
import functools

import jax
import jax.numpy as jnp
from jax import lax
from jax.experimental import pallas as pl
from jax.experimental.pallas import tpu as pltpu
from jax.experimental.pallas import tpu_sc as plsc

N = 10000
D = 128
G = 64
NC, NS = 2, 16
NW = NC * NS
CK = 128
NROWS = 10240
ZR = NROWS // NS
RB = 400
NRB = N // RB

_sc_mesh = plsc.VectorSubcoreMesh(core_axis_name="c", subcore_axis_name="s")


def _deg_body(dsts, ones_r, zrows, out, idx_d, ones_v, acc, nch):
    c = lax.axis_index("c")
    s = lax.axis_index("s")
    pltpu.sync_copy(zrows, acc.at[pl.ds(s * ZR, ZR)])
    pltpu.sync_copy(dsts.at[c, s], idx_d)
    pltpu.sync_copy(ones_r, ones_v)
    plsc.subcore_barrier()

    def chunk(cc, carry):
        pltpu.sync_copy(ones_v, acc.at[idx_d.at[cc]], add=True)
        return carry

    lax.fori_loop(0, nch, chunk, 0)
    plsc.subcore_barrier()
    pltpu.sync_copy(acc.at[pl.ds(s * ZR, ZR)], out.at[c, pl.ds(s * ZR, ZR)])


def _make_deg(nch):
    kern = functools.partial(
        pl.kernel,
        out_type=jax.ShapeDtypeStruct((NC, NROWS, D), jnp.float32),
        mesh=_sc_mesh,
        scratch_types=[
            pltpu.VMEM((nch, CK), jnp.int32),
            pltpu.VMEM((CK, D), jnp.float32),
            pltpu.VMEM_SHARED((NROWS, D), jnp.float32),
        ],
    )
    return kern(functools.partial(_deg_body, nch=nch))


def _make_agg(nch):
    def body(hs, srcs, dsts, zrows, out, idx_s, idx_d, rows, acc, sem):
        c = lax.axis_index("c")
        s = lax.axis_index("s")
        pltpu.sync_copy(zrows, acc.at[pl.ds(s * ZR, ZR)])
        pltpu.sync_copy(srcs.at[c, s], idx_s)
        pltpu.sync_copy(dsts.at[c, s], idx_d)
        plsc.subcore_barrier()

        def chunk(cc, carry):
            pltpu.async_copy(hs.at[idx_s.at[cc]], rows, sem).wait()
            pltpu.sync_copy(rows, acc.at[idx_d.at[cc]], add=True)
            return carry

        lax.fori_loop(0, nch, chunk, 0)
        plsc.subcore_barrier()
        pltpu.sync_copy(acc.at[pl.ds(s * ZR, ZR)], out.at[c, pl.ds(s * ZR, ZR)])

    kern = functools.partial(
        pl.kernel,
        out_type=jax.ShapeDtypeStruct((NC, NROWS, D), jnp.float32),
        mesh=_sc_mesh,
        scratch_types=[
            pltpu.VMEM((nch, CK), jnp.int32),
            pltpu.VMEM((nch, CK), jnp.int32),
            pltpu.VMEM((CK, D), jnp.float32),
            pltpu.VMEM_SHARED((NROWS, D), jnp.float32),
            pltpu.SemaphoreType.DMA,
        ],
    )
    return kern(body)


def _tc1_body(x_ref, w_ref, degp_ref, out_ref):
    deg = degp_ref[0] + degp_ref[1]
    dis = lax.rsqrt(jnp.maximum(deg[:, 0:1], 1.0))
    h = jnp.dot(x_ref[...], w_ref[...], preferred_element_type=jnp.float32)
    out_ref[...] = h * dis


def _tc2_body(aggp_ref, degp_ref, b_ref, w_ref, out_ref):
    deg = degp_ref[0] + degp_ref[1]
    dis = lax.rsqrt(jnp.maximum(deg[:, 0:1], 1.0))
    a = aggp_ref[0] + aggp_ref[1]
    z = jnp.maximum(a * dis + b_ref[...], 0.0)
    h = jnp.dot(z, w_ref[...], preferred_element_type=jnp.float32)
    out_ref[...] = h * dis


def _tc3_body(aggp_ref, degp_ref, b_ref, batch_ref, out_ref, sums, counts):
    i = pl.program_id(0)

    @pl.when(i == 0)
    def _():
        sums[...] = jnp.zeros_like(sums)
        counts[...] = jnp.zeros_like(counts)

    deg = degp_ref[0] + degp_ref[1]
    dis = lax.rsqrt(jnp.maximum(deg[:, 0:1], 1.0))
    a = aggp_ref[0] + aggp_ref[1]
    z = jnp.maximum(a * dis + b_ref[...], 0.0)
    b = batch_ref[0, 0, :]
    oh = (b[:, None] == lax.broadcasted_iota(jnp.int32, (RB, G), 1)
          ).astype(jnp.float32)
    dn = (((0,), (0,)), ((), ()))
    sums[...] += lax.dot_general(oh, z, dn, preferred_element_type=jnp.float32)
    counts[...] += lax.dot_general(oh, jnp.ones((RB, D), jnp.float32), dn,
                                   preferred_element_type=jnp.float32)

    @pl.when(i == NRB - 1)
    def _():
        out_ref[...] = sums[...] / jnp.maximum(counts[...], 1.0)


def kernel(x, edge_index, batch, W1, b1, W2, b2):
    E = edge_index.shape[1]
    ne = E + N
    nch = -(-ne // (NW * CK))
    ep = NW * CK * nch
    npad = ep - ne

    src = edge_index[0].astype(jnp.int32)
    dst = edge_index[1].astype(jnp.int32)
    loop = jnp.arange(N, dtype=jnp.int32)
    padi = jnp.arange(npad, dtype=jnp.int32)
    srcs = jnp.concatenate([src, loop, padi % N]).reshape(NC, NS, nch, CK)
    dsts = jnp.concatenate([dst, loop, N + (padi % (NROWS - N))]
                           ).reshape(NC, NS, nch, CK)

    ones_r = jnp.ones((CK, D), jnp.float32)
    zerosD = jnp.zeros((ZR, D), jnp.float32)

    degp = _make_deg(nch)(dsts, ones_r, zerosD)

    hs1 = pl.pallas_call(
        _tc1_body,
        grid=(NRB,),
        in_specs=[
            pl.BlockSpec((RB, D), lambda i: (i, 0)),
            pl.BlockSpec((D, D), lambda i: (0, 0)),
            pl.BlockSpec((NC, RB, D), lambda i: (0, i, 0)),
        ],
        out_specs=pl.BlockSpec((RB, D), lambda i: (i, 0)),
        out_shape=jax.ShapeDtypeStruct((N, D), jnp.float32),
    )(x, W1, degp)

    agg_fn = _make_agg(nch)
    agg1 = agg_fn(hs1, srcs, dsts, zerosD)

    hs2 = pl.pallas_call(
        _tc2_body,
        grid=(NRB,),
        in_specs=[
            pl.BlockSpec((NC, RB, D), lambda i: (0, i, 0)),
            pl.BlockSpec((NC, RB, D), lambda i: (0, i, 0)),
            pl.BlockSpec((1, D), lambda i: (0, 0)),
            pl.BlockSpec((D, D), lambda i: (0, 0)),
        ],
        out_specs=pl.BlockSpec((RB, D), lambda i: (i, 0)),
        out_shape=jax.ShapeDtypeStruct((N, D), jnp.float32),
    )(agg1, degp, b1.reshape(1, D), W2)

    agg2 = agg_fn(hs2, srcs, dsts, zerosD)

    batch3 = batch.astype(jnp.int32).reshape(NRB, 1, RB)
    out = pl.pallas_call(
        _tc3_body,
        grid=(NRB,),
        in_specs=[
            pl.BlockSpec((NC, RB, D), lambda i: (0, i, 0)),
            pl.BlockSpec((NC, RB, D), lambda i: (0, i, 0)),
            pl.BlockSpec((1, D), lambda i: (0, 0)),
            pl.BlockSpec((1, 1, RB), lambda i: (i, 0, 0)),
        ],
        out_specs=pl.BlockSpec((G, D), lambda i: (0, 0)),
        out_shape=jax.ShapeDtypeStruct((G, D), jnp.float32),
        scratch_shapes=[
            pltpu.VMEM((G, D), jnp.float32),
            pltpu.VMEM((G, D), jnp.float32),
        ],
    )(agg2, degp, b2.reshape(1, D), batch3)

    return out


# speedup vs baseline: 23.3452x; 1.2160x over previous
import functools

import jax
import jax.numpy as jnp
from jax import lax
from jax.experimental import pallas as pl
from jax.experimental.pallas import tpu as pltpu
from jax.experimental.pallas import tpu_sc as plsc

N = 10000
D = 128
G = 64
NC, NS = 2, 16
NW = NC * NS
CK = 128
NROWS = 10240
ZR = NROWS // NS
RB = 400
NRB = N // RB

_sc_mesh = plsc.VectorSubcoreMesh(core_axis_name="c", subcore_axis_name="s")


def _deg_body(dsts, ones_r, zrows, out, idx_d, ones_v, acc, sem0, sem1, nch):
    c = lax.axis_index("c")
    s = lax.axis_index("s")
    pltpu.sync_copy(zrows, acc.at[pl.ds(s * ZR, ZR)])
    pltpu.sync_copy(dsts.at[c, s], idx_d)
    pltpu.sync_copy(ones_r, ones_v)
    plsc.subcore_barrier()
    sems = [sem0, sem1]

    def chunk(cc, carry):
        @pl.when(cc >= 2)
        def _():
            pltpu.make_async_copy(ones_v, acc.at[idx_d.at[cc - 2]],
                                  sems[0]).wait()

        pltpu.async_copy(ones_v, acc.at[idx_d.at[cc]], sems[0], add=True)
        return carry

    lax.fori_loop(0, nch, chunk, 0, unroll=2)
    pltpu.make_async_copy(ones_v, acc.at[idx_d.at[nch - 2]], sems[0]).wait()
    pltpu.make_async_copy(ones_v, acc.at[idx_d.at[nch - 1]], sems[0]).wait()
    plsc.subcore_barrier()
    pltpu.sync_copy(acc.at[pl.ds(s * ZR, ZR)], out.at[c, pl.ds(s * ZR, ZR)])


def _make_deg(nch):
    kern = functools.partial(
        pl.kernel,
        out_type=jax.ShapeDtypeStruct((NC, NROWS, D), jnp.float32),
        mesh=_sc_mesh,
        scratch_types=[
            pltpu.VMEM((nch, CK), jnp.int32),
            pltpu.VMEM((CK, D), jnp.float32),
            pltpu.VMEM_SHARED((NROWS, D), jnp.float32),
            pltpu.SemaphoreType.DMA,
            pltpu.SemaphoreType.DMA,
        ],
    )
    return kern(functools.partial(_deg_body, nch=nch))


def _make_agg(nch):
    def body(hs, srcs, dsts, zrows, out, ibuf, idx_d, rows, acc,
             isem0, isem1, gsem0, gsem1, ssem0, ssem1):
        c = lax.axis_index("c")
        s = lax.axis_index("s")
        pltpu.sync_copy(zrows, acc.at[pl.ds(s * ZR, ZR)])
        pltpu.sync_copy(dsts.at[c, s], idx_d)
        plsc.subcore_barrier()
        isems = [isem0, isem1]
        gsems = [gsem0, gsem1]
        ssems = [ssem0, ssem1]

        pltpu.sync_copy(srcs.at[c, s, 0], ibuf.at[0])
        pltpu.async_copy(srcs.at[c, s, 1], ibuf.at[1], isems[1])
        pltpu.async_copy(hs.at[ibuf.at[0]], rows.at[0], gsems[0])

        def chunk2(ii, carry):
            for b in range(2):
                cc = ii * 2 + b
                nb = 1 - b

                @pl.when(cc < nch)
                def _():
                    pltpu.make_async_copy(hs.at[ibuf.at[b]], rows.at[b],
                                          gsems[b]).wait()
                    pltpu.async_copy(rows.at[b], acc.at[idx_d.at[cc]],
                                     ssems[b], add=True)

                    @pl.when(cc + 2 < nch)
                    def _():
                        pltpu.async_copy(srcs.at[c, s, cc + 2], ibuf.at[b],
                                         isems[b])

                    @pl.when(cc + 1 < nch)
                    def _():
                        @pl.when(cc >= 1)
                        def _():
                            pltpu.make_async_copy(
                                rows.at[nb], acc.at[idx_d.at[cc - 1]],
                                ssems[nb]).wait()

                        pltpu.make_async_copy(srcs.at[c, s, cc + 1],
                                              ibuf.at[nb], isems[nb]).wait()
                        pltpu.async_copy(hs.at[ibuf.at[nb]], rows.at[nb],
                                         gsems[nb])

            return carry

        lax.fori_loop(0, (nch + 1) // 2, chunk2, 0)
        lb = (nch - 1) % 2
        pltpu.make_async_copy(rows.at[lb], acc.at[idx_d.at[nch - 1]],
                              ssems[lb]).wait()
        pltpu.make_async_copy(rows.at[1 - lb], acc.at[idx_d.at[nch - 2]],
                              ssems[1 - lb]).wait()
        plsc.subcore_barrier()
        pltpu.sync_copy(acc.at[pl.ds(s * ZR, ZR)], out.at[c, pl.ds(s * ZR, ZR)])

    kern = functools.partial(
        pl.kernel,
        out_type=jax.ShapeDtypeStruct((NC, NROWS, D), jnp.float32),
        mesh=_sc_mesh,
        scratch_types=[
            pltpu.VMEM((2, CK), jnp.int32),
            pltpu.VMEM((nch, CK), jnp.int32),
            pltpu.VMEM((2, CK, D), jnp.float32),
            pltpu.VMEM_SHARED((NROWS, D), jnp.float32),
            pltpu.SemaphoreType.DMA,
            pltpu.SemaphoreType.DMA,
            pltpu.SemaphoreType.DMA,
            pltpu.SemaphoreType.DMA,
            pltpu.SemaphoreType.DMA,
            pltpu.SemaphoreType.DMA,
        ],
    )
    return kern(body)


def _tc1_body(x_ref, w_ref, degp_ref, out_ref):
    deg = degp_ref[0] + degp_ref[1]
    dis = lax.rsqrt(jnp.maximum(deg[:, 0:1], 1.0))
    h = jnp.dot(x_ref[...], w_ref[...], preferred_element_type=jnp.float32)
    out_ref[...] = h * dis


def _tc2_body(aggp_ref, degp_ref, b_ref, w_ref, out_ref):
    deg = degp_ref[0] + degp_ref[1]
    dis = lax.rsqrt(jnp.maximum(deg[:, 0:1], 1.0))
    a = aggp_ref[0] + aggp_ref[1]
    z = jnp.maximum(a * dis + b_ref[...], 0.0)
    h = jnp.dot(z, w_ref[...], preferred_element_type=jnp.float32)
    out_ref[...] = h * dis


def _tc3_body(aggp_ref, degp_ref, b_ref, batch_ref, out_ref, sums, counts):
    i = pl.program_id(0)

    @pl.when(i == 0)
    def _():
        sums[...] = jnp.zeros_like(sums)
        counts[...] = jnp.zeros_like(counts)

    deg = degp_ref[0] + degp_ref[1]
    dis = lax.rsqrt(jnp.maximum(deg[:, 0:1], 1.0))
    a = aggp_ref[0] + aggp_ref[1]
    z = jnp.maximum(a * dis + b_ref[...], 0.0)
    b = batch_ref[0, 0, :]
    oh = (b[:, None] == lax.broadcasted_iota(jnp.int32, (RB, G), 1)
          ).astype(jnp.float32)
    dn = (((0,), (0,)), ((), ()))
    sums[...] += lax.dot_general(oh, z, dn, preferred_element_type=jnp.float32)
    counts[...] += lax.dot_general(oh, jnp.ones((RB, D), jnp.float32), dn,
                                   preferred_element_type=jnp.float32)

    @pl.when(i == NRB - 1)
    def _():
        out_ref[...] = sums[...] / jnp.maximum(counts[...], 1.0)


def kernel(x, edge_index, batch, W1, b1, W2, b2):
    E = edge_index.shape[1]
    ne = E + N
    nch = -(-ne // (NW * CK))
    ep = NW * CK * nch
    npad = ep - ne

    src = edge_index[0].astype(jnp.int32)
    dst = edge_index[1].astype(jnp.int32)
    loop = jnp.arange(N, dtype=jnp.int32)
    padi = jnp.arange(npad, dtype=jnp.int32)
    srcs = jnp.concatenate([src, loop, padi % N]).reshape(NC, NS, nch, CK)
    dsts = jnp.concatenate([dst, loop, N + (padi % (NROWS - N))]
                           ).reshape(NC, NS, nch, CK)

    ones_r = jnp.ones((CK, D), jnp.float32)
    zerosD = jnp.zeros((ZR, D), jnp.float32)

    degp = _make_deg(nch)(dsts, ones_r, zerosD)

    hs1 = pl.pallas_call(
        _tc1_body,
        grid=(NRB,),
        in_specs=[
            pl.BlockSpec((RB, D), lambda i: (i, 0)),
            pl.BlockSpec((D, D), lambda i: (0, 0)),
            pl.BlockSpec((NC, RB, D), lambda i: (0, i, 0)),
        ],
        out_specs=pl.BlockSpec((RB, D), lambda i: (i, 0)),
        out_shape=jax.ShapeDtypeStruct((N, D), jnp.float32),
    )(x, W1, degp)

    agg_fn = _make_agg(nch)
    agg1 = agg_fn(hs1, srcs, dsts, zerosD)

    hs2 = pl.pallas_call(
        _tc2_body,
        grid=(NRB,),
        in_specs=[
            pl.BlockSpec((NC, RB, D), lambda i: (0, i, 0)),
            pl.BlockSpec((NC, RB, D), lambda i: (0, i, 0)),
            pl.BlockSpec((1, D), lambda i: (0, 0)),
            pl.BlockSpec((D, D), lambda i: (0, 0)),
        ],
        out_specs=pl.BlockSpec((RB, D), lambda i: (i, 0)),
        out_shape=jax.ShapeDtypeStruct((N, D), jnp.float32),
    )(agg1, degp, b1.reshape(1, D), W2)

    agg2 = agg_fn(hs2, srcs, dsts, zerosD)

    batch3 = batch.astype(jnp.int32).reshape(NRB, 1, RB)
    out = pl.pallas_call(
        _tc3_body,
        grid=(NRB,),
        in_specs=[
            pl.BlockSpec((NC, RB, D), lambda i: (0, i, 0)),
            pl.BlockSpec((NC, RB, D), lambda i: (0, i, 0)),
            pl.BlockSpec((1, D), lambda i: (0, 0)),
            pl.BlockSpec((1, 1, RB), lambda i: (i, 0, 0)),
        ],
        out_specs=pl.BlockSpec((G, D), lambda i: (0, 0)),
        out_shape=jax.ShapeDtypeStruct((G, D), jnp.float32),
        scratch_shapes=[
            pltpu.VMEM((G, D), jnp.float32),
            pltpu.VMEM((G, D), jnp.float32),
        ],
    )(agg2, degp, b2.reshape(1, D), batch3)

    return out


# speedup vs baseline: 28.1053x; 1.2039x over previous
import functools

import jax
import jax.numpy as jnp
from jax import lax
from jax.experimental import pallas as pl
from jax.experimental.pallas import tpu as pltpu
from jax.experimental.pallas import tpu_sc as plsc

N = 10000
NP = 10240
D = 128
G = 64
NC, NS = 2, 16
NW = NC * NS
CK = 128
ZR = NP // NS
HB = NP // 128
RB = 512
NRB = NP // RB

_sc_mesh = plsc.VectorSubcoreMesh(core_axis_name="c", subcore_axis_name="s")


def _make_deg(nch):
    def body(dsts, out, idx_d, hist):
        c = lax.axis_index("c")
        s = lax.axis_index("s")
        pltpu.sync_copy(dsts.at[c, s], idx_d)
        zero = jnp.zeros((16,), jnp.float32)

        def zloop(i, carry):
            for j in range(8):
                hist[i, pl.ds(j * 16, 16)] = zero
            return carry

        lax.fori_loop(0, HB, zloop, 0)
        ones = jnp.ones((16,), jnp.float32)

        def hloop(cc, carry):
            for j in range(CK // 16):
                v = idx_d[cc, pl.ds(j * 16, 16)]
                plsc.addupdate_scatter(hist, [v >> 7, v & 127], ones)
            return carry

        lax.fori_loop(0, nch, hloop, 0)
        pltpu.sync_copy(hist, out.at[c, s])

    kern = functools.partial(
        pl.kernel,
        out_type=jax.ShapeDtypeStruct((NC, NS, HB, 128), jnp.float32),
        mesh=_sc_mesh,
        compiler_params=pltpu.CompilerParams(needs_layout_passes=False),
        scratch_types=[
            pltpu.VMEM((nch, CK), jnp.int32),
            pltpu.VMEM((HB, 128), jnp.float32),
        ],
    )
    return kern(body)


def _make_agg(nch):
    def body(hs, srcs, dsts, zrows, out, ibuf, idx_d, rows, acc,
             isem0, isem1, gsem0, gsem1, ssem0, ssem1):
        c = lax.axis_index("c")
        s = lax.axis_index("s")
        pltpu.sync_copy(zrows, acc.at[pl.ds(s * ZR, ZR)])
        pltpu.sync_copy(dsts.at[c, s], idx_d)
        plsc.subcore_barrier()
        isems = [isem0, isem1]
        gsems = [gsem0, gsem1]
        ssems = [ssem0, ssem1]

        pltpu.sync_copy(srcs.at[c, s, 0], ibuf.at[0])
        pltpu.async_copy(srcs.at[c, s, 1], ibuf.at[1], isems[1])
        pltpu.async_copy(hs.at[ibuf.at[0]], rows.at[0], gsems[0])

        def chunk2(ii, carry):
            for b in range(2):
                cc = ii * 2 + b
                nb = 1 - b

                @pl.when(cc < nch)
                def _():
                    pltpu.make_async_copy(hs.at[ibuf.at[b]], rows.at[b],
                                          gsems[b]).wait()
                    pltpu.async_copy(rows.at[b], acc.at[idx_d.at[cc]],
                                     ssems[b], add=True)

                    @pl.when(cc + 2 < nch)
                    def _():
                        pltpu.async_copy(srcs.at[c, s, cc + 2], ibuf.at[b],
                                         isems[b])

                    @pl.when(cc + 1 < nch)
                    def _():
                        @pl.when(cc >= 1)
                        def _():
                            pltpu.make_async_copy(
                                rows.at[nb], acc.at[idx_d.at[cc - 1]],
                                ssems[nb]).wait()

                        pltpu.make_async_copy(srcs.at[c, s, cc + 1],
                                              ibuf.at[nb], isems[nb]).wait()
                        pltpu.async_copy(hs.at[ibuf.at[nb]], rows.at[nb],
                                         gsems[nb])

            return carry

        lax.fori_loop(0, (nch + 1) // 2, chunk2, 0)
        lb = (nch - 1) % 2
        pltpu.make_async_copy(rows.at[lb], acc.at[idx_d.at[nch - 1]],
                              ssems[lb]).wait()
        pltpu.make_async_copy(rows.at[1 - lb], acc.at[idx_d.at[nch - 2]],
                              ssems[1 - lb]).wait()
        plsc.subcore_barrier()
        pltpu.sync_copy(acc.at[pl.ds(s * ZR, ZR)], out.at[c, pl.ds(s * ZR, ZR)])

    kern = functools.partial(
        pl.kernel,
        out_type=jax.ShapeDtypeStruct((NC, NP, D), jnp.float32),
        mesh=_sc_mesh,
        scratch_types=[
            pltpu.VMEM((2, CK), jnp.int32),
            pltpu.VMEM((nch, CK), jnp.int32),
            pltpu.VMEM((2, CK, D), jnp.float32),
            pltpu.VMEM_SHARED((NP, D), jnp.float32),
            pltpu.SemaphoreType.DMA,
            pltpu.SemaphoreType.DMA,
            pltpu.SemaphoreType.DMA,
            pltpu.SemaphoreType.DMA,
            pltpu.SemaphoreType.DMA,
            pltpu.SemaphoreType.DMA,
        ],
    )
    return kern(body)


def _dis_from(degp_ref):
    d2 = degp_ref[0] + degp_ref[1]
    deg = jnp.sum(d2, axis=0)
    return lax.rsqrt(jnp.maximum(deg, 1.0))[:, None]


def _tc1_body(x_ref, w_ref, degp_ref, out_ref):
    dis = _dis_from(degp_ref)
    h = jnp.dot(x_ref[...], w_ref[...], preferred_element_type=jnp.float32)
    out_ref[...] = h * dis


def _tc2_body(aggp_ref, degp_ref, b_ref, w_ref, out_ref):
    dis = _dis_from(degp_ref)
    a = aggp_ref[0] + aggp_ref[1]
    z = jnp.maximum(a * dis + b_ref[...], 0.0)
    h = jnp.dot(z, w_ref[...], preferred_element_type=jnp.float32)
    out_ref[...] = h * dis


def _tc3_body(aggp_ref, degp_ref, b_ref, batch_ref, out_ref, sums, counts):
    i = pl.program_id(0)

    @pl.when(i == 0)
    def _():
        sums[...] = jnp.zeros_like(sums)
        counts[...] = jnp.zeros_like(counts)

    dis = _dis_from(degp_ref)
    a = aggp_ref[0] + aggp_ref[1]
    z = jnp.maximum(a * dis + b_ref[...], 0.0)
    b = batch_ref[0, 0, :]
    oh = (b[:, None] == lax.broadcasted_iota(jnp.int32, (RB, G), 1)
          ).astype(jnp.float32)
    dn = (((0,), (0,)), ((), ()))
    sums[...] += lax.dot_general(oh, z, dn, preferred_element_type=jnp.float32)
    counts[...] += lax.dot_general(oh, jnp.ones((RB, D), jnp.float32), dn,
                                   preferred_element_type=jnp.float32)

    @pl.when(i == NRB - 1)
    def _():
        out_ref[...] = sums[...] / jnp.maximum(counts[...], 1.0)


def kernel(x, edge_index, batch, W1, b1, W2, b2):
    E = edge_index.shape[1]
    ne = E + N
    nch = -(-ne // (NW * CK))
    npad = NW * CK * nch - ne

    src = edge_index[0].astype(jnp.int32)
    dst = edge_index[1].astype(jnp.int32)
    loop = jnp.arange(N, dtype=jnp.int32)
    padi = jnp.arange(npad, dtype=jnp.int32)
    srcs = jnp.concatenate([src, loop, padi % N]).reshape(NC, NS, nch, CK)
    dsts = jnp.concatenate([dst, loop, N + (padi % (NP - N))]
                           ).reshape(NC, NS, nch, CK)

    x_pad = jnp.concatenate([x, jnp.zeros((NP - N, D), x.dtype)])
    batch_pad = jnp.concatenate(
        [batch.astype(jnp.int32), jnp.full((NP - N,), G, jnp.int32)]
    ).reshape(NRB, 1, RB)
    zerosD = jnp.zeros((ZR, D), jnp.float32)

    degp = _make_deg(nch)(dsts).reshape(NC, NS, NP)

    hs1 = pl.pallas_call(
        _tc1_body,
        grid=(NRB,),
        in_specs=[
            pl.BlockSpec((RB, D), lambda i: (i, 0)),
            pl.BlockSpec((D, D), lambda i: (0, 0)),
            pl.BlockSpec((NC, NS, RB), lambda i: (0, 0, i)),
        ],
        out_specs=pl.BlockSpec((RB, D), lambda i: (i, 0)),
        out_shape=jax.ShapeDtypeStruct((NP, D), jnp.float32),
    )(x_pad, W1, degp)

    agg_fn = _make_agg(nch)
    agg1 = agg_fn(hs1, srcs, dsts, zerosD)

    hs2 = pl.pallas_call(
        _tc2_body,
        grid=(NRB,),
        in_specs=[
            pl.BlockSpec((NC, RB, D), lambda i: (0, i, 0)),
            pl.BlockSpec((NC, NS, RB), lambda i: (0, 0, i)),
            pl.BlockSpec((1, D), lambda i: (0, 0)),
            pl.BlockSpec((D, D), lambda i: (0, 0)),
        ],
        out_specs=pl.BlockSpec((RB, D), lambda i: (i, 0)),
        out_shape=jax.ShapeDtypeStruct((NP, D), jnp.float32),
    )(agg1, degp, b1.reshape(1, D), W2)

    agg2 = agg_fn(hs2, srcs, dsts, zerosD)

    out = pl.pallas_call(
        _tc3_body,
        grid=(NRB,),
        in_specs=[
            pl.BlockSpec((NC, RB, D), lambda i: (0, i, 0)),
            pl.BlockSpec((NC, NS, RB), lambda i: (0, 0, i)),
            pl.BlockSpec((1, D), lambda i: (0, 0)),
            pl.BlockSpec((1, 1, RB), lambda i: (i, 0, 0)),
        ],
        out_specs=pl.BlockSpec((G, D), lambda i: (0, 0)),
        out_shape=jax.ShapeDtypeStruct((G, D), jnp.float32),
        scratch_shapes=[
            pltpu.VMEM((G, D), jnp.float32),
            pltpu.VMEM((G, D), jnp.float32),
        ],
    )(agg2, degp, b2.reshape(1, D), batch_pad)

    return out


# speedup vs baseline: 29.4684x; 1.0485x over previous
import functools

import jax
import jax.numpy as jnp
from jax import lax
from jax.experimental import pallas as pl
from jax.experimental.pallas import tpu as pltpu
from jax.experimental.pallas import tpu_sc as plsc

N = 10000
NP = 10240
D = 128
G = 64
NC, NS = 2, 16
NW = NC * NS
CK = 128
ZR = NP // NS
HB = NP // 128
RB = 512
NRB = NP // RB

_sc_mesh = plsc.VectorSubcoreMesh(core_axis_name="c", subcore_axis_name="s")


def _make_deg(nch):
    def body(dsts, out, idx_d, hist):
        c = lax.axis_index("c")
        s = lax.axis_index("s")
        pltpu.sync_copy(dsts.at[c, s], idx_d)
        zero = jnp.zeros((16,), jnp.float32)

        def zloop(i, carry):
            for j in range(8):
                hist[i, pl.ds(j * 16, 16)] = zero
            return carry

        lax.fori_loop(0, HB, zloop, 0)
        ones = jnp.ones((16,), jnp.float32)

        def hloop(cc, carry):
            for j in range(CK // 16):
                v = idx_d[cc, pl.ds(j * 16, 16)]
                plsc.addupdate_scatter(hist, [v >> 7, v & 127], ones)
            return carry

        lax.fori_loop(0, nch, hloop, 0)
        pltpu.sync_copy(hist, out.at[c, s])

    kern = functools.partial(
        pl.kernel,
        out_type=jax.ShapeDtypeStruct((NC, NS, HB, 128), jnp.float32),
        mesh=_sc_mesh,
        compiler_params=pltpu.CompilerParams(needs_layout_passes=False),
        scratch_types=[
            pltpu.VMEM((nch, CK), jnp.int32),
            pltpu.VMEM((HB, 128), jnp.float32),
        ],
    )
    return kern(body)


def _make_agg(nch):
    def body(hs, srcs, dsts, zrows, out, ibuf, idx_d, rows, acc,
             isem0, isem1, gsem0, gsem1, ssem0, ssem1):
        c = lax.axis_index("c")
        s = lax.axis_index("s")

        @pl.when(c == 0)
        def _():
            pltpu.sync_copy(hs.at[pl.ds(s * ZR, ZR)], acc.at[pl.ds(s * ZR, ZR)])

        @pl.when(c == 1)
        def _():
            pltpu.sync_copy(zrows, acc.at[pl.ds(s * ZR, ZR)])

        pltpu.sync_copy(dsts.at[c, s], idx_d)
        plsc.subcore_barrier()
        isems = [isem0, isem1]
        gsems = [gsem0, gsem1]
        ssems = [ssem0, ssem1]

        pltpu.sync_copy(srcs.at[c, s, 0], ibuf.at[0])
        pltpu.async_copy(srcs.at[c, s, 1], ibuf.at[1], isems[1])
        pltpu.async_copy(hs.at[ibuf.at[0]], rows.at[0], gsems[0])

        def chunk2(ii, carry):
            for b in range(2):
                cc = ii * 2 + b
                nb = 1 - b

                @pl.when(cc < nch)
                def _():
                    pltpu.make_async_copy(hs.at[ibuf.at[b]], rows.at[b],
                                          gsems[b]).wait()
                    pltpu.async_copy(rows.at[b], acc.at[idx_d.at[cc]],
                                     ssems[b], add=True)

                    @pl.when(cc + 2 < nch)
                    def _():
                        pltpu.async_copy(srcs.at[c, s, cc + 2], ibuf.at[b],
                                         isems[b])

                    @pl.when(cc + 1 < nch)
                    def _():
                        @pl.when(cc >= 1)
                        def _():
                            pltpu.make_async_copy(
                                rows.at[nb], acc.at[idx_d.at[cc - 1]],
                                ssems[nb]).wait()

                        pltpu.make_async_copy(srcs.at[c, s, cc + 1],
                                              ibuf.at[nb], isems[nb]).wait()
                        pltpu.async_copy(hs.at[ibuf.at[nb]], rows.at[nb],
                                         gsems[nb])

            return carry

        lax.fori_loop(0, (nch + 1) // 2, chunk2, 0)
        lb = (nch - 1) % 2
        pltpu.make_async_copy(rows.at[lb], acc.at[idx_d.at[nch - 1]],
                              ssems[lb]).wait()
        pltpu.make_async_copy(rows.at[1 - lb], acc.at[idx_d.at[nch - 2]],
                              ssems[1 - lb]).wait()
        plsc.subcore_barrier()
        pltpu.sync_copy(acc.at[pl.ds(s * ZR, ZR)], out.at[c, pl.ds(s * ZR, ZR)])

    kern = functools.partial(
        pl.kernel,
        out_type=jax.ShapeDtypeStruct((NC, NP, D), jnp.float32),
        mesh=_sc_mesh,
        scratch_types=[
            pltpu.VMEM((2, CK), jnp.int32),
            pltpu.VMEM((nch, CK), jnp.int32),
            pltpu.VMEM((2, CK, D), jnp.float32),
            pltpu.VMEM_SHARED((NP, D), jnp.float32),
            pltpu.SemaphoreType.DMA,
            pltpu.SemaphoreType.DMA,
            pltpu.SemaphoreType.DMA,
            pltpu.SemaphoreType.DMA,
            pltpu.SemaphoreType.DMA,
            pltpu.SemaphoreType.DMA,
        ],
    )
    return kern(body)


def _dis_from(degp_ref):
    d2 = degp_ref[0] + degp_ref[1]
    deg = jnp.sum(d2, axis=0) + 1.0
    return lax.rsqrt(deg)[:, None]


def _tc1_body(x_ref, w_ref, degp_ref, out_ref):
    dis = _dis_from(degp_ref)
    h = jnp.dot(x_ref[...], w_ref[...], preferred_element_type=jnp.float32)
    out_ref[...] = h * dis


def _tc2_body(aggp_ref, degp_ref, b_ref, w_ref, out_ref):
    dis = _dis_from(degp_ref)
    a = aggp_ref[0] + aggp_ref[1]
    z = jnp.maximum(a * dis + b_ref[...], 0.0)
    h = jnp.dot(z, w_ref[...], preferred_element_type=jnp.float32)
    out_ref[...] = h * dis


def _tc3_body(aggp_ref, degp_ref, b_ref, batch_ref, out_ref, sums, counts):
    i = pl.program_id(0)

    @pl.when(i == 0)
    def _():
        sums[...] = jnp.zeros_like(sums)
        counts[...] = jnp.zeros_like(counts)

    dis = _dis_from(degp_ref)
    a = aggp_ref[0] + aggp_ref[1]
    z = jnp.maximum(a * dis + b_ref[...], 0.0)
    b = batch_ref[0, 0, :]
    oh = (b[:, None] == lax.broadcasted_iota(jnp.int32, (RB, G), 1)
          ).astype(jnp.float32)
    dn = (((0,), (0,)), ((), ()))
    sums[...] += lax.dot_general(oh, z, dn, preferred_element_type=jnp.float32)
    counts[...] += lax.dot_general(oh, jnp.ones((RB, D), jnp.float32), dn,
                                   preferred_element_type=jnp.float32)

    @pl.when(i == NRB - 1)
    def _():
        out_ref[...] = sums[...] / jnp.maximum(counts[...], 1.0)


def kernel(x, edge_index, batch, W1, b1, W2, b2):
    E = edge_index.shape[1]
    nch = -(-E // (NW * CK))
    npad = NW * CK * nch - E

    src = edge_index[0].astype(jnp.int32)
    dst = edge_index[1].astype(jnp.int32)
    padi = jnp.arange(npad, dtype=jnp.int32)
    srcs = jnp.concatenate([src, padi % N]).reshape(NC, NS, nch, CK)
    dsts = jnp.concatenate([dst, N + (padi % (NP - N))]
                           ).reshape(NC, NS, nch, CK)

    x_pad = jnp.concatenate([x, jnp.zeros((NP - N, D), x.dtype)])
    batch_pad = jnp.concatenate(
        [batch.astype(jnp.int32), jnp.full((NP - N,), G, jnp.int32)]
    ).reshape(NRB, 1, RB)
    zerosD = jnp.zeros((ZR, D), jnp.float32)

    degp = _make_deg(nch)(dsts).reshape(NC, NS, NP)

    hs1 = pl.pallas_call(
        _tc1_body,
        grid=(NRB,),
        in_specs=[
            pl.BlockSpec((RB, D), lambda i: (i, 0)),
            pl.BlockSpec((D, D), lambda i: (0, 0)),
            pl.BlockSpec((NC, NS, RB), lambda i: (0, 0, i)),
        ],
        out_specs=pl.BlockSpec((RB, D), lambda i: (i, 0)),
        out_shape=jax.ShapeDtypeStruct((NP, D), jnp.float32),
    )(x_pad, W1, degp)

    agg_fn = _make_agg(nch)
    agg1 = agg_fn(hs1, srcs, dsts, zerosD)

    hs2 = pl.pallas_call(
        _tc2_body,
        grid=(NRB,),
        in_specs=[
            pl.BlockSpec((NC, RB, D), lambda i: (0, i, 0)),
            pl.BlockSpec((NC, NS, RB), lambda i: (0, 0, i)),
            pl.BlockSpec((1, D), lambda i: (0, 0)),
            pl.BlockSpec((D, D), lambda i: (0, 0)),
        ],
        out_specs=pl.BlockSpec((RB, D), lambda i: (i, 0)),
        out_shape=jax.ShapeDtypeStruct((NP, D), jnp.float32),
    )(agg1, degp, b1.reshape(1, D), W2)

    agg2 = agg_fn(hs2, srcs, dsts, zerosD)

    out = pl.pallas_call(
        _tc3_body,
        grid=(NRB,),
        in_specs=[
            pl.BlockSpec((NC, RB, D), lambda i: (0, i, 0)),
            pl.BlockSpec((NC, NS, RB), lambda i: (0, 0, i)),
            pl.BlockSpec((1, D), lambda i: (0, 0)),
            pl.BlockSpec((1, 1, RB), lambda i: (i, 0, 0)),
        ],
        out_specs=pl.BlockSpec((G, D), lambda i: (0, 0)),
        out_shape=jax.ShapeDtypeStruct((G, D), jnp.float32),
        scratch_shapes=[
            pltpu.VMEM((G, D), jnp.float32),
            pltpu.VMEM((G, D), jnp.float32),
        ],
    )(agg2, degp, b2.reshape(1, D), batch_pad)

    return out


# speedup vs baseline: 33.9248x; 1.1512x over previous
import functools

import jax
import jax.numpy as jnp
from jax import lax
from jax.experimental import pallas as pl
from jax.experimental.pallas import tpu as pltpu
from jax.experimental.pallas import tpu_sc as plsc

N = 10000
NP = 10240
D = 128
G = 64
NC, NS = 2, 16
NW = NC * NS
CK = 128
ZR = NP // NS
HB = NP // 128
RB = 512
NRB = NP // RB

_sc_mesh = plsc.VectorSubcoreMesh(core_axis_name="c", subcore_axis_name="s")


def _make_deg(nch):
    def body(dsts, out, idx_d, hist):
        c = lax.axis_index("c")
        s = lax.axis_index("s")
        pltpu.sync_copy(dsts.at[c, s], idx_d)
        zero = jnp.zeros((16,), jnp.float32)

        def zloop(i, carry):
            for j in range(8):
                hist[i, pl.ds(j * 16, 16)] = zero
            return carry

        lax.fori_loop(0, HB, zloop, 0)
        ones = jnp.ones((16,), jnp.float32)

        def hloop(cc, carry):
            for j in range(CK // 16):
                v = idx_d[cc, pl.ds(j * 16, 16)]
                plsc.addupdate_scatter(hist, [v >> 7, v & 127], ones)
            return carry

        lax.fori_loop(0, nch, hloop, 0)
        pltpu.sync_copy(hist, out.at[c, s])

    kern = functools.partial(
        pl.kernel,
        out_type=jax.ShapeDtypeStruct((NC, NS, HB, 128), jnp.float32),
        mesh=_sc_mesh,
        compiler_params=pltpu.CompilerParams(needs_layout_passes=False),
        scratch_types=[
            pltpu.VMEM((nch, CK), jnp.int32),
            pltpu.VMEM((HB, 128), jnp.float32),
        ],
    )
    return kern(body)


def _make_agg(nch):
    def body(hs, srcs, dsts, zrows, out, ibuf, idx_d, rows, acc,
             isem0, isem1, gsem0, gsem1, ssem0, ssem1):
        c = lax.axis_index("c")
        s = lax.axis_index("s")

        @pl.when(c == 0)
        def _():
            pltpu.sync_copy(hs.at[pl.ds(s * ZR, ZR)], acc.at[pl.ds(s * ZR, ZR)])

        @pl.when(c == 1)
        def _():
            pltpu.sync_copy(zrows, acc.at[pl.ds(s * ZR, ZR)])

        pltpu.sync_copy(dsts.at[c, s], idx_d)
        plsc.subcore_barrier()
        isems = [isem0, isem1]
        gsems = [gsem0, gsem1]
        ssems = [ssem0, ssem1]

        pltpu.sync_copy(srcs.at[c, s, 0], ibuf.at[0])
        pltpu.async_copy(srcs.at[c, s, 1], ibuf.at[1], isems[1])
        pltpu.async_copy(hs.at[ibuf.at[0]], rows.at[0], gsems[0])

        def chunk2(ii, carry):
            for b in range(2):
                cc = ii * 2 + b
                nb = 1 - b

                @pl.when(cc < nch)
                def _():
                    @pl.when(cc + 1 < nch)
                    def _():
                        @pl.when(cc >= 1)
                        def _():
                            pltpu.make_async_copy(
                                rows.at[nb], acc.at[idx_d.at[cc - 1]],
                                ssems[nb]).wait()

                        pltpu.make_async_copy(srcs.at[c, s, cc + 1],
                                              ibuf.at[nb], isems[nb]).wait()
                        pltpu.async_copy(hs.at[ibuf.at[nb]], rows.at[nb],
                                         gsems[nb])

                    pltpu.make_async_copy(hs.at[ibuf.at[b]], rows.at[b],
                                          gsems[b]).wait()
                    pltpu.async_copy(rows.at[b], acc.at[idx_d.at[cc]],
                                     ssems[b], add=True)

                    @pl.when(cc + 2 < nch)
                    def _():
                        pltpu.async_copy(srcs.at[c, s, cc + 2], ibuf.at[b],
                                         isems[b])

            return carry

        lax.fori_loop(0, (nch + 1) // 2, chunk2, 0)
        lb = (nch - 1) % 2
        pltpu.make_async_copy(rows.at[lb], acc.at[idx_d.at[nch - 1]],
                              ssems[lb]).wait()
        pltpu.make_async_copy(rows.at[1 - lb], acc.at[idx_d.at[nch - 2]],
                              ssems[1 - lb]).wait()
        plsc.subcore_barrier()
        pltpu.sync_copy(acc.at[pl.ds(s * ZR, ZR)], out.at[c, pl.ds(s * ZR, ZR)])

    kern = functools.partial(
        pl.kernel,
        out_type=jax.ShapeDtypeStruct((NC, NP, D), jnp.float32),
        mesh=_sc_mesh,
        scratch_types=[
            pltpu.VMEM((2, CK), jnp.int32),
            pltpu.VMEM((nch, CK), jnp.int32),
            pltpu.VMEM((2, CK, D), jnp.float32),
            pltpu.VMEM_SHARED((NP, D), jnp.float32),
            pltpu.SemaphoreType.DMA,
            pltpu.SemaphoreType.DMA,
            pltpu.SemaphoreType.DMA,
            pltpu.SemaphoreType.DMA,
            pltpu.SemaphoreType.DMA,
            pltpu.SemaphoreType.DMA,
        ],
    )
    return kern(body)


def _dis_from(degp_ref):
    d2 = degp_ref[0] + degp_ref[1]
    deg = jnp.sum(d2, axis=0) + 1.0
    return lax.rsqrt(deg)[:, None]


def _tc1_body(x_ref, w_ref, degp_ref, out_ref):
    dis = _dis_from(degp_ref)
    h = jnp.dot(x_ref[...], w_ref[...], preferred_element_type=jnp.float32)
    out_ref[...] = h * dis


def _tc2_body(aggp_ref, degp_ref, b_ref, w_ref, out_ref):
    dis = _dis_from(degp_ref)
    a = aggp_ref[0] + aggp_ref[1]
    z = jnp.maximum(a * dis + b_ref[...], 0.0)
    h = jnp.dot(z, w_ref[...], preferred_element_type=jnp.float32)
    out_ref[...] = h * dis


def _tc3_body(aggp_ref, degp_ref, b_ref, batch_ref, out_ref, sums, counts):
    i = pl.program_id(0)

    @pl.when(i == 0)
    def _():
        sums[...] = jnp.zeros_like(sums)
        counts[...] = jnp.zeros_like(counts)

    dis = _dis_from(degp_ref)
    a = aggp_ref[0] + aggp_ref[1]
    z = jnp.maximum(a * dis + b_ref[...], 0.0)
    b = batch_ref[0, 0, :]
    oh = (b[:, None] == lax.broadcasted_iota(jnp.int32, (RB, G), 1)
          ).astype(jnp.float32)
    dn = (((0,), (0,)), ((), ()))
    sums[...] += lax.dot_general(oh, z, dn, preferred_element_type=jnp.float32)
    counts[...] += lax.dot_general(oh, jnp.ones((RB, D), jnp.float32), dn,
                                   preferred_element_type=jnp.float32)

    @pl.when(i == NRB - 1)
    def _():
        out_ref[...] = sums[...] / jnp.maximum(counts[...], 1.0)


def kernel(x, edge_index, batch, W1, b1, W2, b2):
    E = edge_index.shape[1]
    nch = -(-E // (NW * CK))
    npad = NW * CK * nch - E

    src = edge_index[0].astype(jnp.int32)
    dst = edge_index[1].astype(jnp.int32)
    padi = jnp.arange(npad, dtype=jnp.int32)
    srcs = jnp.concatenate([src, padi % N]).reshape(NC, NS, nch, CK)
    dsts = jnp.concatenate([dst, N + (padi % (NP - N))]
                           ).reshape(NC, NS, nch, CK)

    x_pad = jnp.concatenate([x, jnp.zeros((NP - N, D), x.dtype)])
    batch_pad = jnp.concatenate(
        [batch.astype(jnp.int32), jnp.full((NP - N,), G, jnp.int32)]
    ).reshape(NRB, 1, RB)
    zerosD = jnp.zeros((ZR, D), jnp.float32)

    degp = _make_deg(nch)(dsts).reshape(NC, NS, NP)

    hs1 = pl.pallas_call(
        _tc1_body,
        grid=(NRB,),
        in_specs=[
            pl.BlockSpec((RB, D), lambda i: (i, 0)),
            pl.BlockSpec((D, D), lambda i: (0, 0)),
            pl.BlockSpec((NC, NS, RB), lambda i: (0, 0, i)),
        ],
        out_specs=pl.BlockSpec((RB, D), lambda i: (i, 0)),
        out_shape=jax.ShapeDtypeStruct((NP, D), jnp.float32),
    )(x_pad, W1, degp)

    agg_fn = _make_agg(nch)
    agg1 = agg_fn(hs1, srcs, dsts, zerosD)

    hs2 = pl.pallas_call(
        _tc2_body,
        grid=(NRB,),
        in_specs=[
            pl.BlockSpec((NC, RB, D), lambda i: (0, i, 0)),
            pl.BlockSpec((NC, NS, RB), lambda i: (0, 0, i)),
            pl.BlockSpec((1, D), lambda i: (0, 0)),
            pl.BlockSpec((D, D), lambda i: (0, 0)),
        ],
        out_specs=pl.BlockSpec((RB, D), lambda i: (i, 0)),
        out_shape=jax.ShapeDtypeStruct((NP, D), jnp.float32),
    )(agg1, degp, b1.reshape(1, D), W2)

    agg2 = agg_fn(hs2, srcs, dsts, zerosD)

    out = pl.pallas_call(
        _tc3_body,
        grid=(NRB,),
        in_specs=[
            pl.BlockSpec((NC, RB, D), lambda i: (0, i, 0)),
            pl.BlockSpec((NC, NS, RB), lambda i: (0, 0, i)),
            pl.BlockSpec((1, D), lambda i: (0, 0)),
            pl.BlockSpec((1, 1, RB), lambda i: (i, 0, 0)),
        ],
        out_specs=pl.BlockSpec((G, D), lambda i: (0, 0)),
        out_shape=jax.ShapeDtypeStruct((G, D), jnp.float32),
        scratch_shapes=[
            pltpu.VMEM((G, D), jnp.float32),
            pltpu.VMEM((G, D), jnp.float32),
        ],
    )(agg2, degp, b2.reshape(1, D), batch_pad)

    return out


# speedup vs baseline: 33.9430x; 1.0005x over previous
import functools

import jax
import jax.numpy as jnp
from jax import lax
from jax.experimental import pallas as pl
from jax.experimental.pallas import tpu as pltpu
from jax.experimental.pallas import tpu_sc as plsc

N = 10000
NP = 10240
D = 128
G = 64
NC, NS = 2, 16
NW = NC * NS
CK = 128
ZR = NP // NS
HB = NP // 128
RB = 512
NRB = NP // RB

_sc_mesh = plsc.VectorSubcoreMesh(core_axis_name="c", subcore_axis_name="s")


def _make_deg(nch):
    def body(dsts, out, idx_d, hist):
        c = lax.axis_index("c")
        s = lax.axis_index("s")
        pltpu.sync_copy(dsts.at[c, s], idx_d)
        zero = jnp.zeros((16,), jnp.float32)

        def zloop(i, carry):
            for j in range(8):
                hist[i, pl.ds(j * 16, 16)] = zero
            return carry

        lax.fori_loop(0, HB, zloop, 0)
        ones = jnp.ones((16,), jnp.float32)

        def hloop(cc, carry):
            for j in range(CK // 16):
                v = idx_d[cc, pl.ds(j * 16, 16)]
                plsc.addupdate_scatter(hist, [v >> 7, v & 127], ones)
            return carry

        lax.fori_loop(0, nch, hloop, 0)
        pltpu.sync_copy(hist, out.at[c, s])

    kern = functools.partial(
        pl.kernel,
        out_type=jax.ShapeDtypeStruct((NC, NS, HB, 128), jnp.float32),
        mesh=_sc_mesh,
        compiler_params=pltpu.CompilerParams(needs_layout_passes=False),
        scratch_types=[
            pltpu.VMEM((nch, CK), jnp.int32),
            pltpu.VMEM((HB, 128), jnp.float32),
        ],
    )
    return kern(body)


def _make_agg(nch):
    def body(hs, srcs, dsts, zrows, out, ibuf, idx_d, rows, acc,
             isem0, isem1, gsem0, gsem1, ssem0, ssem1):
        c = lax.axis_index("c")
        s = lax.axis_index("s")

        @pl.when(c == 0)
        def _():
            pltpu.sync_copy(hs.at[pl.ds(s * ZR, ZR)], acc.at[pl.ds(s * ZR, ZR)])

        @pl.when(c == 1)
        def _():
            pltpu.sync_copy(zrows, acc.at[pl.ds(s * ZR, ZR)])

        pltpu.sync_copy(dsts.at[c, s], idx_d)
        plsc.subcore_barrier()
        isems = [isem0, isem1]
        gsems = [gsem0, gsem1]
        ssems = [ssem0, ssem1]

        pltpu.sync_copy(srcs.at[c, s, 0], ibuf.at[0])
        pltpu.async_copy(srcs.at[c, s, 1], ibuf.at[1], isems[1])
        pltpu.async_copy(hs.at[ibuf.at[0]], rows.at[0], gsems[0])

        def chunk2(ii, carry):
            for b in range(2):
                cc = ii * 2 + b
                nb = 1 - b

                @pl.when(cc < nch)
                def _():
                    @pl.when(cc + 1 < nch)
                    def _():
                        @pl.when(cc >= 1)
                        def _():
                            pltpu.make_async_copy(
                                rows.at[nb], acc.at[idx_d.at[cc - 1]],
                                ssems[nb]).wait()

                        pltpu.make_async_copy(srcs.at[c, s, cc + 1],
                                              ibuf.at[nb], isems[nb]).wait()
                        pltpu.async_copy(hs.at[ibuf.at[nb, pl.ds(0, CK // 2)]],
                                         rows.at[nb, pl.ds(0, CK // 2)],
                                         gsems[nb])
                        pltpu.async_copy(
                            hs.at[ibuf.at[nb, pl.ds(CK // 2, CK // 2)]],
                            rows.at[nb, pl.ds(CK // 2, CK // 2)], gsems[nb])

                    pltpu.make_async_copy(hs.at[ibuf.at[b]], rows.at[b],
                                          gsems[b]).wait()
                    pltpu.async_copy(rows.at[b], acc.at[idx_d.at[cc]],
                                     ssems[b], add=True)

                    @pl.when(cc + 2 < nch)
                    def _():
                        pltpu.async_copy(srcs.at[c, s, cc + 2], ibuf.at[b],
                                         isems[b])

            return carry

        lax.fori_loop(0, (nch + 1) // 2, chunk2, 0)
        lb = (nch - 1) % 2
        pltpu.make_async_copy(rows.at[lb], acc.at[idx_d.at[nch - 1]],
                              ssems[lb]).wait()
        pltpu.make_async_copy(rows.at[1 - lb], acc.at[idx_d.at[nch - 2]],
                              ssems[1 - lb]).wait()
        plsc.subcore_barrier()
        pltpu.sync_copy(acc.at[pl.ds(s * ZR, ZR)], out.at[c, pl.ds(s * ZR, ZR)])

    kern = functools.partial(
        pl.kernel,
        out_type=jax.ShapeDtypeStruct((NC, NP, D), jnp.float32),
        mesh=_sc_mesh,
        scratch_types=[
            pltpu.VMEM((2, CK), jnp.int32),
            pltpu.VMEM((nch, CK), jnp.int32),
            pltpu.VMEM((2, CK, D), jnp.float32),
            pltpu.VMEM_SHARED((NP, D), jnp.float32),
            pltpu.SemaphoreType.DMA,
            pltpu.SemaphoreType.DMA,
            pltpu.SemaphoreType.DMA,
            pltpu.SemaphoreType.DMA,
            pltpu.SemaphoreType.DMA,
            pltpu.SemaphoreType.DMA,
        ],
    )
    return kern(body)


def _dis_from(degp_ref):
    d2 = degp_ref[0] + degp_ref[1]
    deg = jnp.sum(d2, axis=0) + 1.0
    return lax.rsqrt(deg)[:, None]


def _tc1_body(x_ref, w_ref, degp_ref, out_ref):
    dis = _dis_from(degp_ref)
    h = jnp.dot(x_ref[...], w_ref[...], preferred_element_type=jnp.float32)
    out_ref[...] = h * dis


def _tc2_body(aggp_ref, degp_ref, b_ref, w_ref, out_ref):
    dis = _dis_from(degp_ref)
    a = aggp_ref[0] + aggp_ref[1]
    z = jnp.maximum(a * dis + b_ref[...], 0.0)
    h = jnp.dot(z, w_ref[...], preferred_element_type=jnp.float32)
    out_ref[...] = h * dis


def _tc3_body(aggp_ref, degp_ref, b_ref, batch_ref, out_ref, sums, counts):
    i = pl.program_id(0)

    @pl.when(i == 0)
    def _():
        sums[...] = jnp.zeros_like(sums)
        counts[...] = jnp.zeros_like(counts)

    dis = _dis_from(degp_ref)
    a = aggp_ref[0] + aggp_ref[1]
    z = jnp.maximum(a * dis + b_ref[...], 0.0)
    b = batch_ref[0, 0, :]
    oh = (b[:, None] == lax.broadcasted_iota(jnp.int32, (RB, G), 1)
          ).astype(jnp.float32)
    dn = (((0,), (0,)), ((), ()))
    sums[...] += lax.dot_general(oh, z, dn, preferred_element_type=jnp.float32)
    counts[...] += lax.dot_general(oh, jnp.ones((RB, D), jnp.float32), dn,
                                   preferred_element_type=jnp.float32)

    @pl.when(i == NRB - 1)
    def _():
        out_ref[...] = sums[...] / jnp.maximum(counts[...], 1.0)


def kernel(x, edge_index, batch, W1, b1, W2, b2):
    E = edge_index.shape[1]
    nch = -(-E // (NW * CK))
    npad = NW * CK * nch - E

    src = edge_index[0].astype(jnp.int32)
    dst = edge_index[1].astype(jnp.int32)
    padi = jnp.arange(npad, dtype=jnp.int32)
    srcs = jnp.concatenate([src, padi % N]).reshape(NC, NS, nch, CK)
    dsts = jnp.concatenate([dst, N + (padi % (NP - N))]
                           ).reshape(NC, NS, nch, CK)

    x_pad = jnp.concatenate([x, jnp.zeros((NP - N, D), x.dtype)])
    batch_pad = jnp.concatenate(
        [batch.astype(jnp.int32), jnp.full((NP - N,), G, jnp.int32)]
    ).reshape(NRB, 1, RB)
    zerosD = jnp.zeros((ZR, D), jnp.float32)

    degp = _make_deg(nch)(dsts).reshape(NC, NS, NP)

    hs1 = pl.pallas_call(
        _tc1_body,
        grid=(NRB,),
        in_specs=[
            pl.BlockSpec((RB, D), lambda i: (i, 0)),
            pl.BlockSpec((D, D), lambda i: (0, 0)),
            pl.BlockSpec((NC, NS, RB), lambda i: (0, 0, i)),
        ],
        out_specs=pl.BlockSpec((RB, D), lambda i: (i, 0)),
        out_shape=jax.ShapeDtypeStruct((NP, D), jnp.float32),
    )(x_pad, W1, degp)

    agg_fn = _make_agg(nch)
    agg1 = agg_fn(hs1, srcs, dsts, zerosD)

    hs2 = pl.pallas_call(
        _tc2_body,
        grid=(NRB,),
        in_specs=[
            pl.BlockSpec((NC, RB, D), lambda i: (0, i, 0)),
            pl.BlockSpec((NC, NS, RB), lambda i: (0, 0, i)),
            pl.BlockSpec((1, D), lambda i: (0, 0)),
            pl.BlockSpec((D, D), lambda i: (0, 0)),
        ],
        out_specs=pl.BlockSpec((RB, D), lambda i: (i, 0)),
        out_shape=jax.ShapeDtypeStruct((NP, D), jnp.float32),
    )(agg1, degp, b1.reshape(1, D), W2)

    agg2 = agg_fn(hs2, srcs, dsts, zerosD)

    out = pl.pallas_call(
        _tc3_body,
        grid=(NRB,),
        in_specs=[
            pl.BlockSpec((NC, RB, D), lambda i: (0, i, 0)),
            pl.BlockSpec((NC, NS, RB), lambda i: (0, 0, i)),
            pl.BlockSpec((1, D), lambda i: (0, 0)),
            pl.BlockSpec((1, 1, RB), lambda i: (i, 0, 0)),
        ],
        out_specs=pl.BlockSpec((G, D), lambda i: (0, 0)),
        out_shape=jax.ShapeDtypeStruct((G, D), jnp.float32),
        scratch_shapes=[
            pltpu.VMEM((G, D), jnp.float32),
            pltpu.VMEM((G, D), jnp.float32),
        ],
    )(agg2, degp, b2.reshape(1, D), batch_pad)

    return out
